# baseline (speedup 1.0000x reference)
import jax
import jax.numpy as jnp
from jax.experimental import pallas as pl
from jax.experimental.pallas import tpu as pltpu


def _gcn_body(feat_ref, adj_a_ref, adj_b_ref, w_ref, b_ref, out_ref, s_ref):
    i = pl.program_id(0)

    @pl.when(i == 0)
    def _():
        s_ref[...] = jnp.dot(
            feat_ref[...].astype(jnp.bfloat16),
            w_ref[...].astype(jnp.bfloat16),
            preferred_element_type=jnp.float32,
        ).astype(jnp.bfloat16)

    hb = adj_a_ref.shape[0]
    acc_a = jnp.dot(
        adj_a_ref[...].astype(jnp.bfloat16),
        s_ref[...],
        preferred_element_type=jnp.float32,
    )
    acc_b = jnp.dot(
        adj_b_ref[...].astype(jnp.bfloat16),
        s_ref[...],
        preferred_element_type=jnp.float32,
    )
    out_ref[:hb, :] = acc_a + b_ref[...]
    out_ref[hb:, :] = acc_b + b_ref[...]


def _pick_block(n: int) -> int:
    for ib in (400, 200, 80, 40, 16, 8):
        if n % ib == 0:
            return ib
    return n


@jax.jit
def kernel(feat, adj, W, b):
    N, din = feat.shape
    dout = W.shape[1]
    ib = _pick_block(N)
    hb = ib // 2
    b2 = b.reshape(1, dout)

    grid = (N // ib,)
    out = pl.pallas_call(
        _gcn_body,
        grid=grid,
        in_specs=[
            pl.BlockSpec((N, din), lambda i: (0, 0)),
            pl.BlockSpec((hb, N), lambda i: (2 * i, 0)),
            pl.BlockSpec((hb, N), lambda i: (2 * i + 1, 0)),
            pl.BlockSpec((din, dout), lambda i: (0, 0)),
            pl.BlockSpec((1, dout), lambda i: (0, 0)),
        ],
        out_specs=pl.BlockSpec((ib, dout), lambda i: (i, 0)),
        out_shape=jax.ShapeDtypeStruct((N, dout), jnp.float32),
        scratch_shapes=[pltpu.VMEM((N, dout), jnp.bfloat16)],
        compiler_params=pltpu.CompilerParams(
            vmem_limit_bytes=110 * 1024 * 1024,
        ),
    )(feat, adj, adj, W, b2)
    return out


# speedup vs baseline: 1.0260x; 1.0260x over previous
import jax
import jax.numpy as jnp
from jax.experimental import pallas as pl
from jax.experimental.pallas import tpu as pltpu


def _gcn_body(feat_ref, adj_ref, w_ref, b_ref, out_ref, s_ref):
    i = pl.program_id(0)

    @pl.when(i == 0)
    def _():
        s_ref[...] = jnp.dot(
            feat_ref[...].astype(jnp.bfloat16),
            w_ref[...].astype(jnp.bfloat16),
            preferred_element_type=jnp.float32,
        ).astype(jnp.bfloat16)

    acc = jnp.dot(
        adj_ref[...].astype(jnp.bfloat16),
        s_ref[...],
        preferred_element_type=jnp.float32,
    )
    out_ref[...] = acc + b_ref[...]


def _pick_block(n: int) -> int:
    for ib in (400, 200, 80, 40, 16, 8):
        if n % ib == 0:
            return ib
    return n


@jax.jit
def kernel(feat, adj, W, b):
    N, din = feat.shape
    dout = W.shape[1]
    ib = _pick_block(N)
    b2 = b.reshape(1, dout)

    grid = (N // ib,)
    out = pl.pallas_call(
        _gcn_body,
        grid=grid,
        in_specs=[
            pl.BlockSpec((N, din), lambda i: (0, 0)),
            pl.BlockSpec((ib, N), lambda i: (i, 0)),
            pl.BlockSpec((din, dout), lambda i: (0, 0)),
            pl.BlockSpec((1, dout), lambda i: (0, 0)),
        ],
        out_specs=pl.BlockSpec((ib, dout), lambda i: (i, 0)),
        out_shape=jax.ShapeDtypeStruct((N, dout), jnp.float32),
        scratch_shapes=[pltpu.VMEM((N, dout), jnp.bfloat16)],
        compiler_params=pltpu.CompilerParams(
            vmem_limit_bytes=110 * 1024 * 1024,
        ),
    )(feat, adj, W, b2)
    return out
